# baseline (speedup 1.0000x reference)
import functools

import jax
import jax.numpy as jnp
from jax import lax
from jax.experimental import pallas as pl
from jax.experimental.pallas import tpu as pltpu
from jax.experimental.pallas import tpu_sc as plsc

_NC = 2
_NS = 16
_NW = _NC * _NS
_L = 16
_H = 64


def _make_agg(n, d, nb, k):
  rows_per_tile = -(-n // (_NS * k)) * k
  n_pad = rows_per_tile * _NS
  nz = rows_per_tile // _H
  nsub = 2 * nb
  m = nsub // 3
  mesh = plsc.VectorSubcoreMesh(core_axis_name="c", subcore_axis_name="s")

  @functools.partial(
      pl.kernel,
      out_type=jax.ShapeDtypeStruct((_NC, n_pad, d), jnp.float32),
      mesh=mesh,
      scratch_types=[
          pltpu.VMEM((nb, k), jnp.int32),
          pltpu.VMEM((nb, k), jnp.float32),
          pltpu.VMEM((_H, d), jnp.float32),
          pltpu.VMEM((_H, d), jnp.float32),
          pltpu.VMEM((_H, d), jnp.float32),
          pltpu.VMEM((8, 2 * _H), jnp.int32),
          pltpu.VMEM_SHARED((n_pad, d), jnp.float32),
          pltpu.SemaphoreType.DMA,
          pltpu.SemaphoreType.DMA,
          pltpu.SemaphoreType.DMA,
          pltpu.SemaphoreType.DMA,
          pltpu.SemaphoreType.DMA,
          pltpu.SemaphoreType.DMA,
      ],
  )
  def agg(x_hbm, pk_hbm, w_hbm, out_hbm,
          pk, wv, r0, r1, r2, idx, acc, g0, g1, g2, s0, s1, s2):
    c = lax.axis_index("c")
    s = lax.axis_index("s")
    wid = c * _NS + s
    bufs = (r0, r1, r2)
    gsems = (g0, g1, g2)
    ssems = (s0, s1, s2)

    pltpu.sync_copy(pk_hbm.at[wid], pk)
    pltpu.sync_copy(w_hbm.at[wid], wv)

    def decode(h, j):
      b = h // 2
      off = (h % 2) * _H

      def dec16(g, _):
        v = pk[b, pl.ds(off + g * _L, _L)]
        idx[j, pl.ds(g * _L, _L)] = jnp.bitwise_and(v, jnp.int32(0xFFFF))
        idx[j, pl.ds(_H + g * _L, _L)] = jnp.right_shift(v, jnp.int32(16))
        return 0

      lax.fori_loop(0, _H // _L, dec16, 0)

    def start_gather(j):
      pltpu.async_copy(x_hbm.at[idx.at[j, pl.ds(0, _H)]], bufs[j], gsems[j])

    def wait_gather(j):
      pltpu.make_async_copy(
          x_hbm.at[idx.at[j, pl.ds(0, _H)]], bufs[j], gsems[j]).wait()

    def start_scatter(j):
      pltpu.async_copy(
          bufs[j], acc.at[idx.at[j, pl.ds(_H, _H)]], ssems[j], add=True)

    def wait_scatter(j):
      pltpu.make_async_copy(
          bufs[j], acc.at[idx.at[j, pl.ds(_H, _H)]], ssems[j]).wait()

    def scale(h, j):
      b = h // 2
      off = (h % 2) * _H
      buf = bufs[j]

      def scale16(g, _):
        wvec = wv[b, pl.ds(off + g * _L, _L)]
        for j16 in range(_L):
          w = wvec[j16]
          i = g * _L + j16
          for t in range(d // _L):
            buf[i, pl.ds(t * _L, _L)] = buf[i, pl.ds(t * _L, _L)] * w
        return 0

      lax.fori_loop(0, _H // _L, scale16, 0)

    zero = jnp.zeros((_L,), jnp.float32)

    def zstore(i, _):
      r = i // (d // _L)
      col0 = (i % (d // _L)) * _L
      r0[r, pl.ds(col0, _L)] = zero
      return 0

    lax.fori_loop(0, _H * (d // _L), zstore, 0)
    for t in range(nz):
      pltpu.sync_copy(r0, acc.at[pl.ds(s * rows_per_tile + t * _H, _H)])
    plsc.subcore_barrier()

    decode(0, 0)
    start_gather(0)
    decode(1, 1)
    start_gather(1)

    def body(i, _):
      h0 = 3 * i
      wait_gather(0)
      scale(h0, 0)
      start_scatter(0)

      @pl.when(i >= 1)
      def _():
        wait_scatter(2)
      decode(h0 + 2, 2)
      start_gather(2)

      wait_gather(1)
      scale(h0 + 1, 1)
      start_scatter(1)

      @pl.when(i < m - 1)
      def _():
        wait_scatter(0)
        decode(h0 + 3, 0)
        start_gather(0)

      wait_gather(2)
      scale(h0 + 2, 2)
      start_scatter(2)

      @pl.when(i < m - 1)
      def _():
        wait_scatter(1)
        decode(h0 + 4, 1)
        start_gather(1)

      return 0

    lax.fori_loop(0, m, body, 0)

    wait_scatter(0)
    wait_scatter(1)
    wait_scatter(2)

    plsc.subcore_barrier()
    pltpu.sync_copy(acc.at[pl.ds(s * rows_per_tile, rows_per_tile)],
                    out_hbm.at[c, pl.ds(s * rows_per_tile, rows_per_tile)])

  return agg


def _dense(p, x, W_l, b8, W_r):
  n, d = x.shape
  bn = 2000
  dn = (((1,), (1,)), ((), ()))

  def body(p_ref, x_ref, wl_ref, b_ref, wr_ref, o_ref):
    agg = p_ref[0] + p_ref[1]
    o_ref[...] = (
        lax.dot_general(agg, wl_ref[...], dn,
                        preferred_element_type=jnp.float32,
                        precision=lax.Precision.HIGHEST)
        + lax.dot_general(x_ref[...], wr_ref[...], dn,
                          preferred_element_type=jnp.float32,
                          precision=lax.Precision.HIGHEST)
        + b_ref[0:1, :])

  return pl.pallas_call(
      body,
      grid=(n // bn,),
      in_specs=[
          pl.BlockSpec((2, bn, d), lambda i: (0, i, 0)),
          pl.BlockSpec((bn, d), lambda i: (i, 0)),
          pl.BlockSpec((d, d), lambda i: (0, 0)),
          pl.BlockSpec((8, d), lambda i: (0, 0)),
          pl.BlockSpec((d, d), lambda i: (0, 0)),
      ],
      out_specs=pl.BlockSpec((bn, d), lambda i: (i, 0)),
      out_shape=jax.ShapeDtypeStruct((n, d), jnp.float32),
  )(p, x, W_l, b8, W_r)


_K = 128


def kernel(x, edge_index, edge_weight, W_l, b_l, W_r):
  n, d = x.shape
  e = edge_weight.shape[0]
  nb0 = -(-e // (_NW * _K))
  nb = -(-nb0 // 3) * 3
  pad = _NW * nb * _K - e
  row = jnp.pad(edge_index[0].astype(jnp.int32), (0, pad))
  col = jnp.pad(edge_index[1].astype(jnp.int32), (0, pad))
  packed = jnp.bitwise_or(jnp.left_shift(row, 16), col).reshape(_NW, nb, _K)
  w2 = jnp.pad(edge_weight, (0, pad)).reshape(_NW, nb, _K)
  p = _make_agg(n, d, nb, _K)(x, packed, w2)
  b8 = jnp.broadcast_to(b_l.reshape(1, d), (8, d))
  return _dense(p, x, W_l, b8, W_r)


# speedup vs baseline: 1.0023x; 1.0023x over previous
import functools

import jax
import jax.numpy as jnp
from jax import lax
from jax.experimental import pallas as pl
from jax.experimental.pallas import tpu as pltpu
from jax.experimental.pallas import tpu_sc as plsc

_NC = 2
_NS = 16
_NW = _NC * _NS
_L = 16
_H = 64


def _make_agg(n, d, nb, k):
  rows_per_tile = -(-n // (_NS * k)) * k
  n_pad = rows_per_tile * _NS
  nz = rows_per_tile // _H
  nsub = 2 * nb
  m = nsub // 3
  mesh = plsc.VectorSubcoreMesh(core_axis_name="c", subcore_axis_name="s")

  @functools.partial(
      pl.kernel,
      out_type=jax.ShapeDtypeStruct((_NC, n_pad, d), jnp.float32),
      mesh=mesh,
      scratch_types=[
          pltpu.VMEM((nb, k), jnp.int32),
          pltpu.VMEM((nb, k), jnp.float32),
          pltpu.VMEM((_H, d), jnp.float32),
          pltpu.VMEM((_H, d), jnp.float32),
          pltpu.VMEM((_H, d), jnp.float32),
          pltpu.VMEM((8, 2 * _H), jnp.int32),
          pltpu.VMEM_SHARED((n_pad, d), jnp.float32),
          pltpu.SemaphoreType.DMA,
          pltpu.SemaphoreType.DMA,
          pltpu.SemaphoreType.DMA,
      ],
  )
  def agg(x_hbm, pk_hbm, w_hbm, out_hbm,
          pk, wv, r0, r1, r2, idx, acc, g0, g1, g2):
    c = lax.axis_index("c")
    s = lax.axis_index("s")
    wid = c * _NS + s
    bufs = (r0, r1, r2)
    gsems = (g0, g1, g2)

    pltpu.sync_copy(pk_hbm.at[wid], pk)
    pltpu.sync_copy(w_hbm.at[wid], wv)

    def decode(h, j):
      b = h // 2
      off = (h % 2) * _H

      def dec16(g, _):
        v = pk[b, pl.ds(off + g * _L, _L)]
        idx[j, pl.ds(g * _L, _L)] = jnp.bitwise_and(v, jnp.int32(0xFFFF))
        idx[j, pl.ds(_H + g * _L, _L)] = jnp.right_shift(v, jnp.int32(16))
        return 0

      lax.fori_loop(0, _H // _L, dec16, 0)

    def start_gather(j):
      pltpu.async_copy(x_hbm.at[idx.at[j, pl.ds(0, _H)]], bufs[j], gsems[j])

    def wait_gather(j):
      pltpu.make_async_copy(
          x_hbm.at[idx.at[j, pl.ds(0, _H)]], bufs[j], gsems[j]).wait()

    def scatter(j):
      pltpu.sync_copy(bufs[j], acc.at[idx.at[j, pl.ds(_H, _H)]], add=True)

    def scale(h, j):
      b = h // 2
      off = (h % 2) * _H
      buf = bufs[j]

      def scale16(g, _):
        wvec = wv[b, pl.ds(off + g * _L, _L)]
        for j16 in range(_L):
          w = wvec[j16]
          i = g * _L + j16
          for t in range(d // _L):
            buf[i, pl.ds(t * _L, _L)] = buf[i, pl.ds(t * _L, _L)] * w
        return 0

      lax.fori_loop(0, _H // _L, scale16, 0)

    zero = jnp.zeros((_L,), jnp.float32)

    def zstore(i, _):
      r = i // (d // _L)
      col0 = (i % (d // _L)) * _L
      r0[r, pl.ds(col0, _L)] = zero
      return 0

    lax.fori_loop(0, _H * (d // _L), zstore, 0)
    for t in range(nz):
      pltpu.sync_copy(r0, acc.at[pl.ds(s * rows_per_tile + t * _H, _H)])
    plsc.subcore_barrier()

    for j in range(3):
      decode(j, j)
      start_gather(j)

    def body(i, _):
      h0 = 3 * i
      for u in range(3):
        wait_gather(u)
        scale(h0 + u, u)
        scatter(u)

        @pl.when(i < m - 1)
        def _():
          decode(h0 + u + 3, u)
          start_gather(u)

      return 0

    lax.fori_loop(0, m, body, 0)

    plsc.subcore_barrier()
    pltpu.sync_copy(acc.at[pl.ds(s * rows_per_tile, rows_per_tile)],
                    out_hbm.at[c, pl.ds(s * rows_per_tile, rows_per_tile)])

  return agg


def _dense(p, x, W_l, b8, W_r):
  n, d = x.shape
  bn = 2000
  dn = (((1,), (1,)), ((), ()))

  def body(p_ref, x_ref, wl_ref, b_ref, wr_ref, o_ref):
    agg = p_ref[0] + p_ref[1]
    o_ref[...] = (
        lax.dot_general(agg, wl_ref[...], dn,
                        preferred_element_type=jnp.float32,
                        precision=lax.Precision.HIGHEST)
        + lax.dot_general(x_ref[...], wr_ref[...], dn,
                          preferred_element_type=jnp.float32,
                          precision=lax.Precision.HIGHEST)
        + b_ref[0:1, :])

  return pl.pallas_call(
      body,
      grid=(n // bn,),
      in_specs=[
          pl.BlockSpec((2, bn, d), lambda i: (0, i, 0)),
          pl.BlockSpec((bn, d), lambda i: (i, 0)),
          pl.BlockSpec((d, d), lambda i: (0, 0)),
          pl.BlockSpec((8, d), lambda i: (0, 0)),
          pl.BlockSpec((d, d), lambda i: (0, 0)),
      ],
      out_specs=pl.BlockSpec((bn, d), lambda i: (i, 0)),
      out_shape=jax.ShapeDtypeStruct((n, d), jnp.float32),
  )(p, x, W_l, b8, W_r)


_K = 128


def kernel(x, edge_index, edge_weight, W_l, b_l, W_r):
  n, d = x.shape
  e = edge_weight.shape[0]
  nb0 = -(-e // (_NW * _K))
  nb = -(-nb0 // 3) * 3
  pad = _NW * nb * _K - e
  row = jnp.pad(edge_index[0].astype(jnp.int32), (0, pad))
  col = jnp.pad(edge_index[1].astype(jnp.int32), (0, pad))
  packed = jnp.bitwise_or(jnp.left_shift(row, 16), col).reshape(_NW, nb, _K)
  w2 = jnp.pad(edge_weight, (0, pad)).reshape(_NW, nb, _K)
  p = _make_agg(n, d, nb, _K)(x, packed, w2)
  b8 = jnp.broadcast_to(b_l.reshape(1, d), (8, d))
  return _dense(p, x, W_l, b8, W_r)


# speedup vs baseline: 1.8522x; 1.8479x over previous
import functools

import jax
import jax.numpy as jnp
from jax import lax
from jax.experimental import pallas as pl
from jax.experimental.pallas import tpu as pltpu
from jax.experimental.pallas import tpu_sc as plsc

_NC = 2
_NS = 16
_NW = _NC * _NS
_L = 16
_H = 64


def _make_agg(n, d, nb, k):
  rows_per_tile = -(-n // (_NS * k)) * k
  n_pad = rows_per_tile * _NS
  nz = rows_per_tile // _H
  mesh = plsc.VectorSubcoreMesh(core_axis_name="c", subcore_axis_name="s")

  @functools.partial(
      pl.kernel,
      out_type=jax.ShapeDtypeStruct((_NC, n_pad, d), jnp.float32),
      mesh=mesh,
      scratch_types=[
          pltpu.VMEM((nb, k), jnp.int32),
          pltpu.VMEM((nb, k), jnp.int32),
          pltpu.VMEM((nb, k), jnp.float32),
          pltpu.VMEM((_H, d), jnp.float32),
          pltpu.VMEM((_H, d), jnp.float32),
          pltpu.VMEM_SHARED((n_pad, d), jnp.float32),
          pltpu.SemaphoreType.DMA,
          pltpu.SemaphoreType.DMA,
      ],
  )
  def agg(x_hbm, row_hbm, col_hbm, w_hbm, out_hbm,
          colv, rowv, wv, r0, r1, acc, g0, g1):
    c = lax.axis_index("c")
    s = lax.axis_index("s")
    wid = c * _NS + s

    pltpu.sync_copy(col_hbm.at[wid], colv)
    pltpu.sync_copy(row_hbm.at[wid], rowv)
    pltpu.sync_copy(w_hbm.at[wid], wv)

    def start_gather(b, off, buf, sem):
      pltpu.async_copy(x_hbm.at[colv.at[b, pl.ds(off, _H)]], buf, sem)

    def wait_gather(b, off, buf, sem):
      pltpu.make_async_copy(
          x_hbm.at[colv.at[b, pl.ds(off, _H)]], buf, sem).wait()

    def scale(buf, b, off):
      def scale16(g, _):
        wvec = wv[b, pl.ds(off + g * _L, _L)]
        for j16 in range(_L):
          w = wvec[j16]
          j = g * _L + j16
          for t in range(d // _L):
            buf[j, pl.ds(t * _L, _L)] = buf[j, pl.ds(t * _L, _L)] * w
        return 0

      lax.fori_loop(0, _H // _L, scale16, 0)

    def scatter(buf, b, off):
      pltpu.sync_copy(buf, acc.at[rowv.at[b, pl.ds(off, _H)]], add=True)

    zero = jnp.zeros((_L,), jnp.float32)

    def zstore(i, _):
      r = i // (d // _L)
      col0 = (i % (d // _L)) * _L
      r0[r, pl.ds(col0, _L)] = zero
      return 0

    lax.fori_loop(0, _H * (d // _L), zstore, 0)
    for t in range(nz):
      pltpu.sync_copy(r0, acc.at[pl.ds(s * rows_per_tile + t * _H, _H)])
    plsc.subcore_barrier()

    start_gather(0, 0, r0, g0)

    def body(b, _):
      start_gather(b, _H, r1, g1)
      wait_gather(b, 0, r0, g0)
      scale(r0, b, 0)
      scatter(r0, b, 0)

      @pl.when(b + 1 < nb)
      def _():
        start_gather(b + 1, 0, r0, g0)

      wait_gather(b, _H, r1, g1)
      scale(r1, b, _H)
      scatter(r1, b, _H)
      return 0

    lax.fori_loop(0, nb, body, 0)

    plsc.subcore_barrier()
    pltpu.sync_copy(acc.at[pl.ds(s * rows_per_tile, rows_per_tile)],
                    out_hbm.at[c, pl.ds(s * rows_per_tile, rows_per_tile)])

  return agg


def _dense(p, x, W_l, b8, W_r):
  n, d = x.shape
  bn = 2000
  dn = (((1,), (1,)), ((), ()))

  def body(p_ref, x_ref, wl_ref, b_ref, wr_ref, o_ref):
    agg = p_ref[0] + p_ref[1]
    o_ref[...] = (
        lax.dot_general(agg, wl_ref[...], dn,
                        preferred_element_type=jnp.float32,
                        precision=lax.Precision.HIGHEST)
        + lax.dot_general(x_ref[...], wr_ref[...], dn,
                          preferred_element_type=jnp.float32,
                          precision=lax.Precision.HIGHEST)
        + b_ref[0:1, :])

  return pl.pallas_call(
      body,
      grid=(n // bn,),
      in_specs=[
          pl.BlockSpec((2, bn, d), lambda i: (0, i, 0)),
          pl.BlockSpec((bn, d), lambda i: (i, 0)),
          pl.BlockSpec((d, d), lambda i: (0, 0)),
          pl.BlockSpec((8, d), lambda i: (0, 0)),
          pl.BlockSpec((d, d), lambda i: (0, 0)),
      ],
      out_specs=pl.BlockSpec((bn, d), lambda i: (i, 0)),
      out_shape=jax.ShapeDtypeStruct((n, d), jnp.float32),
  )(p, x, W_l, b8, W_r)


_K = 128


def kernel(x, edge_index, edge_weight, W_l, b_l, W_r):
  n, d = x.shape
  e = edge_weight.shape[0]
  nb = -(-e // (_NW * _K))
  pad = _NW * nb * _K - e
  row = jnp.pad(edge_index[0].astype(jnp.int32), (0, pad)).reshape(_NW, nb, _K)
  col = jnp.pad(edge_index[1].astype(jnp.int32), (0, pad)).reshape(_NW, nb, _K)
  w2 = jnp.pad(edge_weight, (0, pad)).reshape(_NW, nb, _K)
  p = _make_agg(n, d, nb, _K)(x, row, col, w2)
  b8 = jnp.broadcast_to(b_l.reshape(1, d), (8, d))
  return _dense(p, x, W_l, b8, W_r)


# speedup vs baseline: 1.8623x; 1.0054x over previous
import functools

import jax
import jax.numpy as jnp
from jax import lax
from jax.experimental import pallas as pl
from jax.experimental.pallas import tpu as pltpu
from jax.experimental.pallas import tpu_sc as plsc

_NC = 2
_NS = 16
_NW = _NC * _NS
_L = 16
_H = 64


def _make_agg(n, d, nb, k):
  rows_per_tile = -(-n // (_NS * k)) * k
  n_pad = rows_per_tile * _NS
  nz = rows_per_tile // _H
  mesh = plsc.VectorSubcoreMesh(core_axis_name="c", subcore_axis_name="s")

  @functools.partial(
      pl.kernel,
      out_type=jax.ShapeDtypeStruct((_NC, n_pad, d), jnp.float32),
      mesh=mesh,
      scratch_types=[
          pltpu.VMEM((nb, k), jnp.int32),
          pltpu.VMEM((nb, k), jnp.int32),
          pltpu.VMEM((nb, k), jnp.float32),
          pltpu.VMEM((_H, d), jnp.float32),
          pltpu.VMEM((_H, d), jnp.float32),
          pltpu.VMEM_SHARED((n_pad, d), jnp.float32),
          pltpu.SemaphoreType.DMA,
          pltpu.SemaphoreType.DMA,
      ],
  )
  def agg(x_hbm, row_hbm, col_hbm, w_hbm, out_hbm,
          colv, rowv, wv, r0, r1, acc, g0, g1):
    c = lax.axis_index("c")
    s = lax.axis_index("s")
    wid = c * _NS + s

    pltpu.sync_copy(col_hbm.at[wid], colv)
    pltpu.sync_copy(row_hbm.at[wid], rowv)
    pltpu.sync_copy(w_hbm.at[wid], wv)

    def start_gather(b, off, buf, sem):
      pltpu.async_copy(x_hbm.at[colv.at[b, pl.ds(off, _H)]], buf, sem)

    def wait_gather(b, off, buf, sem):
      pltpu.make_async_copy(
          x_hbm.at[colv.at[b, pl.ds(off, _H)]], buf, sem).wait()

    def scale(buf, b, off):
      for g in range(_H // _L):
        wvec = wv[b, pl.ds(off + g * _L, _L)]
        for j16 in range(_L):
          w = wvec[j16]
          j = g * _L + j16
          for t in range(d // _L):
            buf[j, pl.ds(t * _L, _L)] = buf[j, pl.ds(t * _L, _L)] * w

    def scatter(buf, b, off):
      pltpu.sync_copy(buf, acc.at[rowv.at[b, pl.ds(off, _H)]], add=True)

    zero = jnp.zeros((_L,), jnp.float32)

    def zstore(i, _):
      r = i // (d // _L)
      col0 = (i % (d // _L)) * _L
      r0[r, pl.ds(col0, _L)] = zero
      return 0

    lax.fori_loop(0, _H * (d // _L), zstore, 0)
    for t in range(nz):
      pltpu.sync_copy(r0, acc.at[pl.ds(s * rows_per_tile + t * _H, _H)])
    plsc.subcore_barrier()

    start_gather(0, 0, r0, g0)

    def body(b, _):
      start_gather(b, _H, r1, g1)
      wait_gather(b, 0, r0, g0)
      scale(r0, b, 0)
      scatter(r0, b, 0)

      @pl.when(b + 1 < nb)
      def _():
        start_gather(b + 1, 0, r0, g0)

      wait_gather(b, _H, r1, g1)
      scale(r1, b, _H)
      scatter(r1, b, _H)
      return 0

    lax.fori_loop(0, nb, body, 0)

    plsc.subcore_barrier()
    pltpu.sync_copy(acc.at[pl.ds(s * rows_per_tile, rows_per_tile)],
                    out_hbm.at[c, pl.ds(s * rows_per_tile, rows_per_tile)])

  return agg


_DN = (((1,), (1,)), ((), ()))


def _dense_r(x, W_r, b8):
  n, d = x.shape
  bn = 2000

  def body(x_ref, wr_ref, b_ref, o_ref):
    o_ref[...] = lax.dot_general(
        x_ref[...], wr_ref[...], _DN,
        preferred_element_type=jnp.float32,
        precision=lax.Precision.HIGHEST) + b_ref[0:1, :]

  return pl.pallas_call(
      body,
      grid=(n // bn,),
      in_specs=[
          pl.BlockSpec((bn, d), lambda i: (i, 0)),
          pl.BlockSpec((d, d), lambda i: (0, 0)),
          pl.BlockSpec((8, d), lambda i: (0, 0)),
      ],
      out_specs=pl.BlockSpec((bn, d), lambda i: (i, 0)),
      out_shape=jax.ShapeDtypeStruct((n, d), jnp.float32),
  )(x, W_r, b8)


def _dense_l(p, xr, W_l):
  n, d = xr.shape
  bn = 2000

  def body(p_ref, xr_ref, wl_ref, o_ref):
    agg = p_ref[0] + p_ref[1]
    o_ref[...] = lax.dot_general(
        agg, wl_ref[...], _DN,
        preferred_element_type=jnp.float32,
        precision=lax.Precision.HIGHEST) + xr_ref[...]

  return pl.pallas_call(
      body,
      grid=(n // bn,),
      in_specs=[
          pl.BlockSpec((2, bn, d), lambda i: (0, i, 0)),
          pl.BlockSpec((bn, d), lambda i: (i, 0)),
          pl.BlockSpec((d, d), lambda i: (0, 0)),
      ],
      out_specs=pl.BlockSpec((bn, d), lambda i: (i, 0)),
      out_shape=jax.ShapeDtypeStruct((n, d), jnp.float32),
  )(p, xr, W_l)


_K = 128


def kernel(x, edge_index, edge_weight, W_l, b_l, W_r):
  n, d = x.shape
  e = edge_weight.shape[0]
  nb = -(-e // (_NW * _K))
  pad = _NW * nb * _K - e
  row = jnp.pad(edge_index[0].astype(jnp.int32), (0, pad)).reshape(_NW, nb, _K)
  col = jnp.pad(edge_index[1].astype(jnp.int32), (0, pad)).reshape(_NW, nb, _K)
  w2 = jnp.pad(edge_weight, (0, pad)).reshape(_NW, nb, _K)
  b8 = jnp.broadcast_to(b_l.reshape(1, d), (8, d))
  xr = _dense_r(x, W_r, b8)
  p = _make_agg(n, d, nb, _K)(x, row, col, w2)
  return _dense_l(p, xr, W_l)
